# baseline (speedup 1.0000x reference)
import functools

import jax
import jax.numpy as jnp
from jax import lax
from jax.experimental import pallas as pl
from jax.experimental.pallas import tpu as pltpu
from jax.experimental.pallas import tpu_sc as plsc

N = 10000
E = 320000
D = 128

NC = 2
NS = 16
L = 16
NW = NC * NS
EPW = E // NW
C = 40
HC0 = 24
HC1 = 16
NCH = EPW // C
DC = 128
EPT = E // NS
NDC = EPT // DC
DTAIL = EPT - NDC * DC
NPT = 624
NTAIL = N - NS * NPT

_MESH = plsc.VectorSubcoreMesh(
    core_axis_name="c", subcore_axis_name="s", num_cores=NC, num_subcores=NS
)


def _rsqrt16(d):
    i = lax.bitcast_convert_type(d, jnp.int32)
    i = jnp.int32(0x5F3759DF) - lax.shift_right_logical(i, 1)
    y = lax.bitcast_convert_type(i, jnp.float32)
    for _ in range(3):
        y = y * (1.5 - 0.5 * d * y * y)
    return y


@functools.partial(
    pl.kernel,
    out_type=[
        jax.ShapeDtypeStruct((NC, N, D), jnp.float32),
        jax.ShapeDtypeStruct((N,), jnp.float32),
    ],
    mesh=_MESH,
    compiler_params=pltpu.CompilerParams(needs_layout_passes=False),
    scratch_types=[
        pltpu.VMEM_SHARED((N, D), jnp.float32),
        pltpu.VMEM_SHARED((N,), jnp.float32),
        pltpu.VMEM((C,), jnp.int32),
        pltpu.VMEM((C,), jnp.int32),
        pltpu.VMEM((C,), jnp.int32),
        pltpu.VMEM((C,), jnp.int32),
        pltpu.VMEM((C,), jnp.float32),
        pltpu.VMEM((C,), jnp.float32),
        pltpu.VMEM((DC,), jnp.int32),
        pltpu.VMEM((DC,), jnp.int32),
        pltpu.VMEM((DTAIL,), jnp.int32),
        pltpu.VMEM((DC,), jnp.float32),
        pltpu.VMEM((HC0, D), jnp.float32),
        pltpu.VMEM((HC1, D), jnp.float32),
        pltpu.VMEM((C, D), jnp.float32),
        pltpu.VMEM((C, D), jnp.float32),
        pltpu.VMEM((624,), jnp.float32),
        pltpu.SemaphoreType.DMA,
        pltpu.SemaphoreType.DMA,
        pltpu.SemaphoreType.DMA,
        pltpu.SemaphoreType.DMA,
        pltpu.SemaphoreType.DMA,
        pltpu.SemaphoreType.DMA,
        pltpu.SemaphoreType.DMA,
        pltpu.SemaphoreType.DMA,
        pltpu.SemaphoreType.DMA,
        pltpu.SemaphoreType.DMA,
        pltpu.SemaphoreType.DMA,
        pltpu.SemaphoreType.DMA,
        pltpu.SemaphoreType.DMA,
        pltpu.SemaphoreType.DMA,
    ],
)
def _sc_edge_pass(
    h_hbm, row_hbm, col_hbm, eattr_hbm, aggr_hbm, dinv_hbm,
    aggr_s, deg_s, rowiA, rowiB, coliA, coliB, scaleA, scaleB,
    degiA, degiB, degiT, ones_v, eh0, eh1, msgA, msgB, dbuf,
    semIA, semIB, semHA, semHB, semDA, semDB, semE0, semE1, semPA, semPB,
    semSA, semSB, semCA, semCB,
):
    cid = lax.axis_index("c")
    sid = lax.axis_index("s")
    wid = sid * NC + cid

    zero16 = jnp.zeros((L,), jnp.float32)

    def _zmsg(j, _):
        for k in range(D // L):
            msgA[j, pl.ds(k * L, L)] = zero16
        return 0

    lax.fori_loop(0, C, _zmsg, 0)

    for k in range(DC // L):
        ones_v[pl.ds(k * L, L)] = zero16 + 1.0
    for g in range(624 // L):
        dbuf[pl.ds(g * L, L)] = zero16

    r0 = sid * NPT
    for i in range(NPT // C):
        pltpu.sync_copy(msgA, aggr_s.at[pl.ds(r0 + i * C, C)])
    rem = NPT % C
    pltpu.sync_copy(msgA.at[pl.ds(0, rem)],
                    aggr_s.at[pl.ds(r0 + (NPT // C) * C, rem)])
    pltpu.sync_copy(dbuf, deg_s.at[pl.ds(r0, NPT)])

    @pl.when(sid == 0)
    def _():
        pltpu.sync_copy(msgA.at[pl.ds(0, NTAIL)],
                        aggr_s.at[pl.ds(NS * NPT, NTAIL)])
        pltpu.sync_copy(dbuf.at[pl.ds(0, NTAIL)],
                        deg_s.at[pl.ds(NS * NPT, NTAIL)])

    plsc.subcore_barrier()

    tbase = sid * EPT

    def _deg_chunk(i, degiX, semPX, degiY, semPY):
        nxt = jnp.minimum(i + 1, NDC - 1)
        pltpu.async_copy(row_hbm.at[pl.ds(tbase + nxt * DC, DC)], degiY, semPY)
        pltpu.make_async_copy(row_hbm.at[pl.ds(0, DC)], degiX, semPX).wait()
        pltpu.sync_copy(ones_v, deg_s.at[degiX], add=True)

    pltpu.async_copy(row_hbm.at[pl.ds(tbase, DC)], degiA, semPA)

    def _deg_pair(t, _):
        _deg_chunk(2 * t, degiA, semPA, degiB, semPB)
        _deg_chunk(2 * t + 1, degiB, semPB, degiA, semPA)
        return 0

    lax.fori_loop(0, NDC // 2, _deg_pair, 0)
    pltpu.make_async_copy(row_hbm.at[pl.ds(0, DC)], degiA, semPA).wait()
    pltpu.async_copy(row_hbm.at[pl.ds(tbase + NDC * DC, DTAIL)], degiT, semPA).wait()
    pltpu.sync_copy(ones_v.at[pl.ds(0, DTAIL)], deg_s.at[degiT], add=True)

    plsc.subcore_barrier()

    pltpu.sync_copy(deg_s.at[pl.ds(r0, NPT)], dbuf)

    def _rs_body(g, _):
        s = pl.ds(g * L, L)
        dbuf[s] = _rsqrt16(dbuf[s] + 1.0)
        return 0

    lax.fori_loop(0, NPT // L, _rs_body, 0)
    pltpu.sync_copy(dbuf, dinv_hbm.at[pl.ds(r0, NPT)])

    @pl.when(sid == 0)
    def _():
        pltpu.sync_copy(deg_s.at[pl.ds(NS * NPT, NTAIL)], dbuf.at[pl.ds(0, NTAIL)])
        dbuf[pl.ds(0, L)] = _rsqrt16(dbuf[pl.ds(0, L)] + 1.0)
        pltpu.sync_copy(dbuf.at[pl.ds(0, NTAIL)], dinv_hbm.at[pl.ds(NS * NPT, NTAIL)])

    plsc.subcore_barrier()

    wbase = wid * NCH

    def _issue_rowi(ch2, rowiX, semIX):
        eb = (wbase + ch2) * C
        pltpu.async_copy(row_hbm.at[pl.ds(eb, C)], rowiX, semIX)

    def _issue_coli(ch1, coliY, semCY):
        eb = (wbase + ch1) * C
        pltpu.async_copy(col_hbm.at[pl.ds(eb, C)], coliY, semCY)

    def _issue_gathers(rowiX, msgX, scaleX, semHX, semDX):
        pltpu.async_copy(h_hbm.at[rowiX], msgX, semHX)
        pltpu.async_copy(dinv_hbm.at[rowiX], scaleX, semDX)

    def _issue_eattr(nxt, half):
        eb = (wbase + nxt) * C + (0 if half == 0 else HC0)
        if half == 0:
            pltpu.async_copy(eattr_hbm.at[pl.ds(eb, HC0)], eh0, semE0)
        else:
            pltpu.async_copy(eattr_hbm.at[pl.ds(eb, HC1)], eh1, semE1)

    def _half_compute(msgX, scaleX, ehX, j0, nrows):
        def _edge(j2, _):
            j = j0 + j2
            nb = plsc.load_gather(scaleX, [jnp.zeros((L,), jnp.int32) + j])
            for k in range(D // L):
                s = pl.ds(k * L, L)
                msgX[j, s] = jnp.maximum(msgX[j, s] + ehX[j2, s], 0.0) * nb
            return 0

        lax.fori_loop(0, nrows, _edge, 0, unroll=4)

    def _chunk(ch, cur, nxtb, first=False):
        rowiX, coliX, scaleX, msgX, semIX, semCX, semHX, semDX, semSX = cur
        rowiY, coliY, scaleY, msgY, semIY, semCY, semHY, semDY, semSY = nxtb
        nxt = jnp.minimum(ch + 1, NCH - 1)
        nxt2 = jnp.minimum(ch + 2, NCH - 1)
        pltpu.make_async_copy(h_hbm.at[rowiX], msgX, semHX).wait()
        pltpu.make_async_copy(dinv_hbm.at[rowiX], scaleX, semDX).wait()
        pltpu.make_async_copy(row_hbm.at[pl.ds(0, C)], rowiY, semIY).wait()
        if not first:
            pltpu.make_async_copy(msgY, aggr_s.at[coliY], semSY).wait()
        _issue_rowi(nxt2, rowiX, semIX)
        _issue_gathers(rowiY, msgY, scaleY, semHY, semDY)
        _issue_coli(nxt, coliY, semCY)
        pltpu.make_async_copy(eattr_hbm.at[pl.ds(0, HC0)], eh0, semE0).wait()
        _half_compute(msgX, scaleX, eh0, 0, HC0)
        _issue_eattr(nxt, 0)
        pltpu.make_async_copy(eattr_hbm.at[pl.ds(0, HC1)], eh1, semE1).wait()
        _half_compute(msgX, scaleX, eh1, HC0, HC1)
        pltpu.make_async_copy(col_hbm.at[pl.ds(0, C)], coliX, semCX).wait()
        pltpu.async_copy(msgX, aggr_s.at[coliX], semSX, add=True)
        _issue_eattr(nxt, 1)

    bufA = (rowiA, coliA, scaleA, msgA, semIA, semCA, semHA, semDA, semSA)
    bufB = (rowiB, coliB, scaleB, msgB, semIB, semCB, semHB, semDB, semSB)

    pltpu.async_copy(row_hbm.at[pl.ds(wbase * C, C)], rowiA, semIA)
    pltpu.make_async_copy(row_hbm.at[pl.ds(0, C)], rowiA, semIA).wait()
    _issue_rowi(1, rowiB, semIB)
    _issue_gathers(rowiA, msgA, scaleA, semHA, semDA)
    _issue_coli(0, coliA, semCA)
    _issue_eattr(0, 0)
    _issue_eattr(0, 1)

    _chunk(0, bufA, bufB, first=True)
    _chunk(1, bufB, bufA)

    def _pair(t, _):
        _chunk(2 * t, bufA, bufB)
        _chunk(2 * t + 1, bufB, bufA)
        return 0

    lax.fori_loop(1, NCH // 2, _pair, 0)

    pltpu.make_async_copy(row_hbm.at[pl.ds(0, C)], rowiB, semIB).wait()
    pltpu.make_async_copy(h_hbm.at[rowiA], msgA, semHA).wait()
    pltpu.make_async_copy(dinv_hbm.at[rowiA], scaleA, semDA).wait()
    pltpu.make_async_copy(col_hbm.at[pl.ds(0, C)], coliA, semCA).wait()
    pltpu.make_async_copy(eattr_hbm.at[pl.ds(0, HC0)], eh0, semE0).wait()
    pltpu.make_async_copy(eattr_hbm.at[pl.ds(0, HC1)], eh1, semE1).wait()
    pltpu.make_async_copy(msgB, aggr_s.at[coliB], semSB).wait()

    plsc.subcore_barrier()

    for i in range(NPT // C):
        pltpu.sync_copy(aggr_s.at[pl.ds(r0 + i * C, C)], msgA)
        pltpu.sync_copy(msgA, aggr_hbm.at[cid, pl.ds(r0 + i * C, C)])
    pltpu.sync_copy(aggr_s.at[pl.ds(r0 + (NPT // C) * C, rem)],
                    msgA.at[pl.ds(0, rem)])
    pltpu.sync_copy(msgA.at[pl.ds(0, rem)],
                    aggr_hbm.at[cid, pl.ds(r0 + (NPT // C) * C, rem)])

    @pl.when(sid == 0)
    def _():
        pltpu.sync_copy(aggr_s.at[pl.ds(NS * NPT, NTAIL)], msgA.at[pl.ds(0, NTAIL)])
        pltpu.sync_copy(msgA.at[pl.ds(0, NTAIL)],
                        aggr_hbm.at[cid, pl.ds(NS * NPT, NTAIL)])


_RB = 1000


def _mm_body(x_ref, w_ref, b_ref, h_ref):
    xr = jnp.maximum(x_ref[...], 0.0)
    h_ref[...] = (
        lax.dot_general(xr, w_ref[...], (((1,), (1,)), ((), ())),
                        preferred_element_type=jnp.float32)
        + b_ref[...]
    )


def _comb_body(a_ref, h_ref, r_ref, dv_ref, out_ref, st_ref):
    i = pl.program_id(0)
    hr = jnp.maximum(h_ref[...] + r_ref[...], 0.0)
    a = a_ref[...]
    dv = dv_ref[...]
    o = (a[0] + a[1]) * dv + hr * (dv * dv)
    out_ref[...] = o

    @pl.when(i == 0)
    def _():
        st_ref[...] = jnp.zeros_like(st_ref)

    st_ref[0:1] = st_ref[0:1] + jnp.sum(o, axis=0, keepdims=True)
    st_ref[1:2] = st_ref[1:2] + jnp.sum(o * o, axis=0, keepdims=True)


def _norm_body(o_ref, st_ref, g_ref, b_ref, y_ref):
    st = st_ref[...]
    mean = st[0:1] * (1.0 / N)
    var = st[1:2] * (1.0 / N) - mean * mean
    sc = g_ref[...] * lax.rsqrt(var + 1e-5)
    sh = b_ref[...] - mean * sc
    y_ref[...] = o_ref[...] * sc + sh


def kernel(x, edge_index, edge_attr, W, b, root_emb, gamma, beta):
    row = edge_index[0].astype(jnp.int32)
    col = edge_index[1].astype(jnp.int32)

    h = pl.pallas_call(
        _mm_body,
        grid=(N // _RB,),
        in_specs=[
            pl.BlockSpec((_RB, D), lambda i: (i, 0)),
            pl.BlockSpec((D, D), lambda i: (0, 0)),
            pl.BlockSpec((1, D), lambda i: (0, 0)),
        ],
        out_specs=pl.BlockSpec((_RB, D), lambda i: (i, 0)),
        out_shape=jax.ShapeDtypeStruct((N, D), jnp.float32),
    )(x, W, b.reshape(1, D))

    aggr2, dinv = _sc_edge_pass(h, row, col, edge_attr)

    out, st = pl.pallas_call(
        _comb_body,
        grid=(N // _RB,),
        in_specs=[
            pl.BlockSpec((NC, _RB, D), lambda i: (0, i, 0)),
            pl.BlockSpec((_RB, D), lambda i: (i, 0)),
            pl.BlockSpec((1, D), lambda i: (0, 0)),
            pl.BlockSpec((_RB, 1), lambda i: (i, 0)),
        ],
        out_specs=[
            pl.BlockSpec((_RB, D), lambda i: (i, 0)),
            pl.BlockSpec((8, D), lambda i: (0, 0)),
        ],
        out_shape=[
            jax.ShapeDtypeStruct((N, D), jnp.float32),
            jax.ShapeDtypeStruct((8, D), jnp.float32),
        ],
    )(aggr2, h, root_emb, dinv.reshape(N, 1))

    y = pl.pallas_call(
        _norm_body,
        grid=(N // _RB,),
        in_specs=[
            pl.BlockSpec((_RB, D), lambda i: (i, 0)),
            pl.BlockSpec((8, D), lambda i: (0, 0)),
            pl.BlockSpec((1, D), lambda i: (0, 0)),
            pl.BlockSpec((1, D), lambda i: (0, 0)),
        ],
        out_specs=pl.BlockSpec((_RB, D), lambda i: (i, 0)),
        out_shape=jax.ShapeDtypeStruct((N, D), jnp.float32),
    )(out, st, gamma.reshape(1, D), beta.reshape(1, D))

    return y


# speedup vs baseline: 1.4878x; 1.4878x over previous
import functools

import jax
import jax.numpy as jnp
from jax import lax
from jax.experimental import pallas as pl
from jax.experimental.pallas import tpu as pltpu
from jax.experimental.pallas import tpu_sc as plsc

N = 10000
E = 320000
D = 128

NC = 2
NS = 16
L = 16
NW = NC * NS
EPW = E // NW
C = 40
HC0 = 24
HC1 = 16
NCH = EPW // C
DC = 128
EPT = E // NS
NDC = EPT // DC
DTAIL = EPT - NDC * DC
NPT = 624
NTAIL = N - NS * NPT

_MESH = plsc.VectorSubcoreMesh(
    core_axis_name="c", subcore_axis_name="s", num_cores=NC, num_subcores=NS
)


def _rsqrt16(d):
    i = lax.bitcast_convert_type(d, jnp.int32)
    i = jnp.int32(0x5F3759DF) - lax.shift_right_logical(i, 1)
    y = lax.bitcast_convert_type(i, jnp.float32)
    for _ in range(3):
        y = y * (1.5 - 0.5 * d * y * y)
    return y


@functools.partial(
    pl.kernel,
    out_type=[
        jax.ShapeDtypeStruct((NC, N, D), jnp.float32),
        jax.ShapeDtypeStruct((N,), jnp.float32),
    ],
    mesh=_MESH,
    compiler_params=pltpu.CompilerParams(needs_layout_passes=False),
    scratch_types=[
        pltpu.VMEM_SHARED((N, D), jnp.float32),
        pltpu.VMEM_SHARED((N,), jnp.float32),
        pltpu.VMEM((C,), jnp.int32),
        pltpu.VMEM((C,), jnp.int32),
        pltpu.VMEM((C,), jnp.int32),
        pltpu.VMEM((C,), jnp.int32),
        pltpu.VMEM((C,), jnp.float32),
        pltpu.VMEM((C,), jnp.float32),
        pltpu.VMEM((DC,), jnp.int32),
        pltpu.VMEM((DC,), jnp.int32),
        pltpu.VMEM((DTAIL,), jnp.int32),
        pltpu.VMEM((DC,), jnp.float32),
        pltpu.VMEM((HC0, D), jnp.float32),
        pltpu.VMEM((HC1, D), jnp.float32),
        pltpu.VMEM((C, D), jnp.float32),
        pltpu.VMEM((C, D), jnp.float32),
        pltpu.VMEM((624,), jnp.float32),
        pltpu.SemaphoreType.DMA,
        pltpu.SemaphoreType.DMA,
        pltpu.SemaphoreType.DMA,
        pltpu.SemaphoreType.DMA,
        pltpu.SemaphoreType.DMA,
        pltpu.SemaphoreType.DMA,
        pltpu.SemaphoreType.DMA,
        pltpu.SemaphoreType.DMA,
        pltpu.SemaphoreType.DMA,
        pltpu.SemaphoreType.DMA,
        pltpu.SemaphoreType.DMA,
        pltpu.SemaphoreType.DMA,
        pltpu.SemaphoreType.DMA,
        pltpu.SemaphoreType.DMA,
    ],
)
def _sc_edge_pass(
    h_hbm, row_hbm, col_hbm, eattr_hbm, aggr_hbm, dinv_hbm,
    aggr_s, deg_s, rowiA, rowiB, coliA, coliB, scaleA, scaleB,
    degiA, degiB, degiT, ones_v, eh0, eh1, msgA, msgB, dbuf,
    semIA, semIB, semHA, semHB, semDA, semDB, semE0, semE1, semPA, semPB,
    semSA, semSB, semCA, semCB,
):
    cid = lax.axis_index("c")
    sid = lax.axis_index("s")
    wid = sid * NC + cid

    zero16 = jnp.zeros((L,), jnp.float32)

    def _zmsg(j, _):
        for k in range(D // L):
            msgA[j, pl.ds(k * L, L)] = zero16
        return 0

    lax.fori_loop(0, C, _zmsg, 0)

    for k in range(DC // L):
        ones_v[pl.ds(k * L, L)] = zero16 + 1.0
    for g in range(624 // L):
        dbuf[pl.ds(g * L, L)] = zero16

    r0 = sid * NPT
    for i in range(NPT // C):
        pltpu.sync_copy(msgA, aggr_s.at[pl.ds(r0 + i * C, C)])
    rem = NPT % C
    pltpu.sync_copy(msgA.at[pl.ds(0, rem)],
                    aggr_s.at[pl.ds(r0 + (NPT // C) * C, rem)])
    pltpu.sync_copy(dbuf, deg_s.at[pl.ds(r0, NPT)])

    @pl.when(sid == 0)
    def _():
        pltpu.sync_copy(msgA.at[pl.ds(0, NTAIL)],
                        aggr_s.at[pl.ds(NS * NPT, NTAIL)])
        pltpu.sync_copy(dbuf.at[pl.ds(0, NTAIL)],
                        deg_s.at[pl.ds(NS * NPT, NTAIL)])

    plsc.subcore_barrier()

    tbase = sid * EPT

    def _deg_chunk(i, degiX, semPX, degiY, semPY):
        nxt = jnp.minimum(i + 1, NDC - 1)
        pltpu.async_copy(row_hbm.at[pl.ds(tbase + nxt * DC, DC)], degiY, semPY)
        pltpu.make_async_copy(row_hbm.at[pl.ds(0, DC)], degiX, semPX).wait()
        pltpu.sync_copy(ones_v, deg_s.at[degiX], add=True)

    pltpu.async_copy(row_hbm.at[pl.ds(tbase, DC)], degiA, semPA)

    def _deg_pair(t, _):
        _deg_chunk(2 * t, degiA, semPA, degiB, semPB)
        _deg_chunk(2 * t + 1, degiB, semPB, degiA, semPA)
        return 0

    lax.fori_loop(0, NDC // 2, _deg_pair, 0)
    pltpu.make_async_copy(row_hbm.at[pl.ds(0, DC)], degiA, semPA).wait()
    pltpu.async_copy(row_hbm.at[pl.ds(tbase + NDC * DC, DTAIL)], degiT, semPA).wait()
    pltpu.sync_copy(ones_v.at[pl.ds(0, DTAIL)], deg_s.at[degiT], add=True)

    plsc.subcore_barrier()

    pltpu.sync_copy(deg_s.at[pl.ds(r0, NPT)], dbuf)

    def _rs_body(g, _):
        s = pl.ds(g * L, L)
        dbuf[s] = _rsqrt16(dbuf[s] + 1.0)
        return 0

    lax.fori_loop(0, NPT // L, _rs_body, 0)
    pltpu.sync_copy(dbuf, dinv_hbm.at[pl.ds(r0, NPT)])

    @pl.when(sid == 0)
    def _():
        pltpu.sync_copy(deg_s.at[pl.ds(NS * NPT, NTAIL)], dbuf.at[pl.ds(0, NTAIL)])
        dbuf[pl.ds(0, L)] = _rsqrt16(dbuf[pl.ds(0, L)] + 1.0)
        pltpu.sync_copy(dbuf.at[pl.ds(0, NTAIL)], dinv_hbm.at[pl.ds(NS * NPT, NTAIL)])

    plsc.subcore_barrier()

    wbase = wid * NCH

    def _issue_rowi(ch2, rowiX, semIX):
        eb = (wbase + ch2) * C
        pltpu.async_copy(row_hbm.at[pl.ds(eb, C)], rowiX, semIX)

    def _issue_coli(ch1, coliY, semCY):
        eb = (wbase + ch1) * C
        pltpu.async_copy(col_hbm.at[pl.ds(eb, C)], coliY, semCY)

    def _issue_gathers(rowiX, msgX, scaleX, semHX, semDX):
        pltpu.async_copy(h_hbm.at[rowiX], msgX, semHX)
        pltpu.async_copy(dinv_hbm.at[rowiX], scaleX, semDX)

    def _issue_eattr(nxt, half):
        eb = (wbase + nxt) * C + (0 if half == 0 else HC0)
        if half == 0:
            pltpu.async_copy(eattr_hbm.at[pl.ds(eb, HC0)], eh0, semE0)
        else:
            pltpu.async_copy(eattr_hbm.at[pl.ds(eb, HC1)], eh1, semE1)

    def _half_compute(msgX, scaleX, ehX, j0, nrows, static=False):
        if static:
            for j2 in range(nrows):
                j = j0 + j2
                nb = plsc.load_gather(scaleX, [jnp.full((L,), j, jnp.int32)])
                for k in range(D // L):
                    s = pl.ds(k * L, L)
                    msgX[j, s] = jnp.maximum(msgX[j, s] + ehX[j2, s], 0.0) * nb
            return

        def _edge(j2, _):
            j = j0 + j2
            nb = plsc.load_gather(scaleX, [jnp.zeros((L,), jnp.int32) + j])
            for k in range(D // L):
                s = pl.ds(k * L, L)
                msgX[j, s] = jnp.maximum(msgX[j, s] + ehX[j2, s], 0.0) * nb
            return 0

        lax.fori_loop(0, nrows, _edge, 0)

    def _chunk(ch, cur, nxtb, first=False, static=False):
        rowiX, coliX, scaleX, msgX, semIX, semCX, semHX, semDX, semSX = cur
        rowiY, coliY, scaleY, msgY, semIY, semCY, semHY, semDY, semSY = nxtb
        nxt = jnp.minimum(ch + 1, NCH - 1)
        nxt2 = jnp.minimum(ch + 2, NCH - 1)
        pltpu.make_async_copy(h_hbm.at[rowiX], msgX, semHX).wait()
        pltpu.make_async_copy(dinv_hbm.at[rowiX], scaleX, semDX).wait()
        pltpu.make_async_copy(row_hbm.at[pl.ds(0, C)], rowiY, semIY).wait()
        if not first:
            pltpu.make_async_copy(msgY, aggr_s.at[coliY], semSY).wait()
        _issue_rowi(nxt2, rowiX, semIX)
        _issue_gathers(rowiY, msgY, scaleY, semHY, semDY)
        _issue_coli(nxt, coliY, semCY)
        pltpu.make_async_copy(eattr_hbm.at[pl.ds(0, HC0)], eh0, semE0).wait()
        _half_compute(msgX, scaleX, eh0, 0, HC0, static=static)
        _issue_eattr(nxt, 0)
        pltpu.make_async_copy(eattr_hbm.at[pl.ds(0, HC1)], eh1, semE1).wait()
        _half_compute(msgX, scaleX, eh1, HC0, HC1, static=static)
        pltpu.make_async_copy(col_hbm.at[pl.ds(0, C)], coliX, semCX).wait()
        pltpu.async_copy(msgX, aggr_s.at[coliX], semSX, add=True)
        _issue_eattr(nxt, 1)

    bufA = (rowiA, coliA, scaleA, msgA, semIA, semCA, semHA, semDA, semSA)
    bufB = (rowiB, coliB, scaleB, msgB, semIB, semCB, semHB, semDB, semSB)

    pltpu.async_copy(row_hbm.at[pl.ds(wbase * C, C)], rowiA, semIA)
    pltpu.make_async_copy(row_hbm.at[pl.ds(0, C)], rowiA, semIA).wait()
    _issue_rowi(1, rowiB, semIB)
    _issue_gathers(rowiA, msgA, scaleA, semHA, semDA)
    _issue_coli(0, coliA, semCA)
    _issue_eattr(0, 0)
    _issue_eattr(0, 1)

    _chunk(0, bufA, bufB, first=True)
    _chunk(1, bufB, bufA)

    def _pair(t, _):
        _chunk(2 * t, bufA, bufB, static=True)
        _chunk(2 * t + 1, bufB, bufA, static=True)
        return 0

    lax.fori_loop(1, NCH // 2, _pair, 0)

    pltpu.make_async_copy(row_hbm.at[pl.ds(0, C)], rowiB, semIB).wait()
    pltpu.make_async_copy(h_hbm.at[rowiA], msgA, semHA).wait()
    pltpu.make_async_copy(dinv_hbm.at[rowiA], scaleA, semDA).wait()
    pltpu.make_async_copy(col_hbm.at[pl.ds(0, C)], coliA, semCA).wait()
    pltpu.make_async_copy(eattr_hbm.at[pl.ds(0, HC0)], eh0, semE0).wait()
    pltpu.make_async_copy(eattr_hbm.at[pl.ds(0, HC1)], eh1, semE1).wait()
    pltpu.make_async_copy(msgB, aggr_s.at[coliB], semSB).wait()

    plsc.subcore_barrier()

    for i in range(NPT // C):
        pltpu.sync_copy(aggr_s.at[pl.ds(r0 + i * C, C)], msgA)
        pltpu.sync_copy(msgA, aggr_hbm.at[cid, pl.ds(r0 + i * C, C)])
    pltpu.sync_copy(aggr_s.at[pl.ds(r0 + (NPT // C) * C, rem)],
                    msgA.at[pl.ds(0, rem)])
    pltpu.sync_copy(msgA.at[pl.ds(0, rem)],
                    aggr_hbm.at[cid, pl.ds(r0 + (NPT // C) * C, rem)])

    @pl.when(sid == 0)
    def _():
        pltpu.sync_copy(aggr_s.at[pl.ds(NS * NPT, NTAIL)], msgA.at[pl.ds(0, NTAIL)])
        pltpu.sync_copy(msgA.at[pl.ds(0, NTAIL)],
                        aggr_hbm.at[cid, pl.ds(NS * NPT, NTAIL)])


_RB = 1000


def _mm_body(x_ref, w_ref, b_ref, h_ref):
    xr = jnp.maximum(x_ref[...], 0.0)
    h_ref[...] = (
        lax.dot_general(xr, w_ref[...], (((1,), (1,)), ((), ())),
                        preferred_element_type=jnp.float32)
        + b_ref[...]
    )


def _comb_body(a_ref, h_ref, r_ref, dv_ref, out_ref, st_ref):
    i = pl.program_id(0)
    hr = jnp.maximum(h_ref[...] + r_ref[...], 0.0)
    a = a_ref[...]
    dv = dv_ref[...]
    o = (a[0] + a[1]) * dv + hr * (dv * dv)
    out_ref[...] = o

    @pl.when(i == 0)
    def _():
        st_ref[...] = jnp.zeros_like(st_ref)

    st_ref[0:1] = st_ref[0:1] + jnp.sum(o, axis=0, keepdims=True)
    st_ref[1:2] = st_ref[1:2] + jnp.sum(o * o, axis=0, keepdims=True)


def _norm_body(o_ref, st_ref, g_ref, b_ref, y_ref):
    st = st_ref[...]
    mean = st[0:1] * (1.0 / N)
    var = st[1:2] * (1.0 / N) - mean * mean
    sc = g_ref[...] * lax.rsqrt(var + 1e-5)
    sh = b_ref[...] - mean * sc
    y_ref[...] = o_ref[...] * sc + sh


def kernel(x, edge_index, edge_attr, W, b, root_emb, gamma, beta):
    row = edge_index[0].astype(jnp.int32)
    col = edge_index[1].astype(jnp.int32)

    h = pl.pallas_call(
        _mm_body,
        grid=(N // _RB,),
        in_specs=[
            pl.BlockSpec((_RB, D), lambda i: (i, 0)),
            pl.BlockSpec((D, D), lambda i: (0, 0)),
            pl.BlockSpec((1, D), lambda i: (0, 0)),
        ],
        out_specs=pl.BlockSpec((_RB, D), lambda i: (i, 0)),
        out_shape=jax.ShapeDtypeStruct((N, D), jnp.float32),
    )(x, W, b.reshape(1, D))

    aggr2, dinv = _sc_edge_pass(h, row, col, edge_attr)

    out, st = pl.pallas_call(
        _comb_body,
        grid=(N // _RB,),
        in_specs=[
            pl.BlockSpec((NC, _RB, D), lambda i: (0, i, 0)),
            pl.BlockSpec((_RB, D), lambda i: (i, 0)),
            pl.BlockSpec((1, D), lambda i: (0, 0)),
            pl.BlockSpec((_RB, 1), lambda i: (i, 0)),
        ],
        out_specs=[
            pl.BlockSpec((_RB, D), lambda i: (i, 0)),
            pl.BlockSpec((8, D), lambda i: (0, 0)),
        ],
        out_shape=[
            jax.ShapeDtypeStruct((N, D), jnp.float32),
            jax.ShapeDtypeStruct((8, D), jnp.float32),
        ],
    )(aggr2, h, root_emb, dinv.reshape(N, 1))

    y = pl.pallas_call(
        _norm_body,
        grid=(N // _RB,),
        in_specs=[
            pl.BlockSpec((_RB, D), lambda i: (i, 0)),
            pl.BlockSpec((8, D), lambda i: (0, 0)),
            pl.BlockSpec((1, D), lambda i: (0, 0)),
            pl.BlockSpec((1, D), lambda i: (0, 0)),
        ],
        out_specs=pl.BlockSpec((_RB, D), lambda i: (i, 0)),
        out_shape=jax.ShapeDtypeStruct((N, D), jnp.float32),
    )(out, st, gamma.reshape(1, D), beta.reshape(1, D))

    return y
